# baseline (speedup 1.0000x reference)
import functools

import jax
import jax.numpy as jnp
from jax import lax
from jax.experimental import pallas as pl
from jax.experimental.pallas import tpu as pltpu
from jax.experimental.pallas import tpu_sc as plsc

B = 16384
D = 32
NC = 2
NS = 16
NW = NC * NS
BPW = B // NW
CHUNKS = BPW // 16


def kernel(user_id, item_id, user_table, item_table):
    ut = user_table.T
    it = item_table.T

    mesh = plsc.VectorSubcoreMesh(core_axis_name="c", subcore_axis_name="s")

    @functools.partial(
        pl.kernel,
        mesh=mesh,
        out_type=jax.ShapeDtypeStruct((B,), jnp.float32),
        compiler_params=pltpu.CompilerParams(
            needs_layout_passes=False, use_tc_tiling_on_sc=False),
        scratch_types=[
            pltpu.VMEM((BPW,), jnp.int32),
            pltpu.VMEM((BPW,), jnp.int32),
            pltpu.VMEM((D, BPW), jnp.float32),
            pltpu.VMEM((D, BPW), jnp.float32),
            pltpu.VMEM((BPW,), jnp.float32),
            pltpu.SemaphoreType.DMA,
        ],
    )
    def run(uid_hbm, iid_hbm, ut_hbm, it_hbm, out_hbm,
            uidx_v, iidx_v, ucols_v, icols_v, out_v, sem):
        wid = lax.axis_index("s") * NC + lax.axis_index("c")
        base = wid * BPW

        pltpu.sync_copy(uid_hbm.at[pl.ds(base, BPW)], uidx_v)
        pltpu.sync_copy(iid_hbm.at[pl.ds(base, BPW)], iidx_v)

        copies = []
        for d in range(D):
            copies.append(pltpu.async_copy(
                ut_hbm.at[d].at[uidx_v], ucols_v.at[d], sem))
            copies.append(pltpu.async_copy(
                it_hbm.at[d].at[iidx_v], icols_v.at[d], sem))
        for c in copies:
            c.wait()

        def chunk_body(c, carry):
            off = c * 16
            acc = jnp.zeros((16,), jnp.float32)
            for d in range(D):
                acc = acc + (ucols_v[d, pl.ds(off, 16)]
                             * icols_v[d, pl.ds(off, 16)])
            out_v[pl.ds(off, 16)] = acc
            return carry

        lax.fori_loop(0, CHUNKS, chunk_body, 0)

        pltpu.sync_copy(out_v, out_hbm.at[pl.ds(base, BPW)])

    return run(user_id, item_id, ut, it)


# speedup vs baseline: 5.7271x; 5.7271x over previous
import functools

import jax
import jax.numpy as jnp
from jax import lax
from jax.experimental import pallas as pl
from jax.experimental.pallas import tpu as pltpu
from jax.experimental.pallas import tpu_sc as plsc

B = 16384
D = 32
NC = 2
NS = 16
NW = NC * NS
BPW = B // NW
CHUNKS = BPW // 16


def kernel(user_id, item_id, user_table, item_table):
    mesh = plsc.VectorSubcoreMesh(core_axis_name="c", subcore_axis_name="s")

    @functools.partial(
        pl.kernel,
        mesh=mesh,
        out_type=jax.ShapeDtypeStruct((B,), jnp.float32),
        compiler_params=pltpu.CompilerParams(
            needs_layout_passes=False, use_tc_tiling_on_sc=False),
        scratch_types=[
            pltpu.VMEM((BPW,), jnp.int32),
            pltpu.VMEM((BPW,), jnp.int32),
            pltpu.VMEM((BPW, D), jnp.float32),
            pltpu.VMEM((BPW, D), jnp.float32),
            pltpu.VMEM((BPW,), jnp.float32),
            pltpu.SemaphoreType.DMA,
        ],
    )
    def run(uid_hbm, iid_hbm, ut_hbm, it_hbm, out_hbm,
            uidx_v, iidx_v, urows_v, irows_v, out_v, sem):
        wid = lax.axis_index("s") * NC + lax.axis_index("c")
        base = wid * BPW

        pltpu.sync_copy(uid_hbm.at[pl.ds(base, BPW)], uidx_v)
        pltpu.sync_copy(iid_hbm.at[pl.ds(base, BPW)], iidx_v)

        cu = pltpu.async_copy(ut_hbm.at[uidx_v], urows_v, sem)
        ci = pltpu.async_copy(it_hbm.at[iidx_v], irows_v, sem)
        cu.wait()
        ci.wait()

        lanes = lax.iota(jnp.int32, 16)

        def chunk_body(c, carry):
            rows = c * 16 + lanes
            acc = jnp.zeros((16,), jnp.float32)
            for h in range(D // 16):
                for s in range(16):
                    cols = ((lanes + s) & 15) + 16 * h
                    uv = plsc.load_gather(urows_v, [rows, cols])
                    iv = plsc.load_gather(irows_v, [rows, cols])
                    acc = acc + uv * iv
            out_v[pl.ds(c * 16, 16)] = acc
            return carry

        lax.fori_loop(0, CHUNKS, chunk_body, 0)

        pltpu.sync_copy(out_v, out_hbm.at[pl.ds(base, BPW)])

    return run(user_id, item_id, user_table, item_table)
